# Initial kernel scaffold; baseline (speedup 1.0000x reference)
import jax
import jax.numpy as jnp
from jax.experimental import pallas as pl


def kernel(x_ncl, w_packed, b_packed):
    raise NotImplementedError("write your pallas kernel here")


# speedup vs baseline: 2.7622x; 2.7622x over previous
import jax
import jax.numpy as jnp
from jax import lax
from jax.experimental import pallas as pl
from jax.experimental.pallas import tpu as pltpu

_LAYER_DIMS = ((3, 16), (16, 32), (32, 16), (16, 3))
_RELUS = (True, True, True, False)
_K = 3
_LANE = 128

_S = 4
_B = 32


def _align(v, m):
    return -(-v // m) * m


def _build_weights(w_packed, b_packed):
    ws, bs = [], []
    eye = jnp.eye(_S, dtype=jnp.float32)
    for li, (cin, cout) in enumerate(_LAYER_DIMS):
        wt = w_packed[3 * li:3 * li + 3, :cout, :cin]
        seg = _align(_S * cin, 16)
        w5 = jnp.einsum('pq,toc->potqc', eye, wt)
        w = w5.reshape(_S * cout, 3, _S * cin)
        w = jnp.pad(w, ((0, 0), (0, 0), (0, seg - _S * cin)))
        w = w.reshape(_S * cout, 3 * seg)
        m_pad = _align(_S * cout, 16)
        if m_pad != _S * cout:
            w = jnp.pad(w, ((0, m_pad - _S * cout), (0, 0)))
        b = jnp.tile(b_packed[li, :cout, :], (_S, 1))
        if m_pad != _S * cout:
            b = jnp.pad(b, ((0, m_pad - _S * cout), (0, 0)))
        ws.append(w.astype(jnp.bfloat16))
        bs.append(b)
    return ws, bs


def _make_body(l_valid, l_pad, n_groups):
    rows_in = _S * _LAYER_DIMS[0][0]
    seg0 = _align(rows_in, 16)

    def body(x_ref, w1, w2, w3, w4, b1, b2, b3, b4, o_ref):
        wrefs = (w1, w2, w3, w4)
        brefs = (b1, b2, b3, b4)
        lane = lax.broadcasted_iota(jnp.int32, (1, l_pad), 1)
        keep_prev = lane >= 1
        keep_next = lane < (l_valid - 1)

        for g in range(n_groups):
            h = x_ref[0, g * rows_in:(g + 1) * rows_in, :]
            if seg0 != rows_in:
                h = jnp.concatenate(
                    [h, jnp.zeros((seg0 - rows_in, l_pad), h.dtype)], axis=0)
            h = h.astype(jnp.bfloat16)
            for li, (cin, cout) in enumerate(_LAYER_DIMS):
                zero = jnp.bfloat16(0.0)
                prev = jnp.where(keep_prev, pltpu.roll(h, 1, axis=1), zero)
                nxt = jnp.where(keep_next, pltpu.roll(h, l_pad - 1, axis=1), zero)
                stk = jnp.concatenate([prev, h, nxt], axis=0)
                y = jnp.dot(wrefs[li][...], stk,
                            preferred_element_type=jnp.float32)
                y = y + brefs[li][...]
                if _RELUS[li]:
                    h = jnp.maximum(y, 0.0).astype(jnp.bfloat16)
                else:
                    o_ref[0, g * rows_in:(g + 1) * rows_in, :] = \
                        y[:rows_in, :]
    return body


def kernel(x_ncl, w_packed, b_packed):
    n, c, l = x_ncl.shape
    l_pad = _align(max(l, _LANE), _LANE)
    n_pad = _align(n, _B)

    xp = x_ncl
    if n_pad != n or l_pad != l:
        xp = jnp.pad(x_ncl, ((0, n_pad - n), (0, 0), (0, l_pad - l)))

    n_steps = n_pad // _B
    n_groups = _B // _S
    rows = _B * c
    xr = xp.reshape(n_steps, rows, l_pad)

    ws, bs = _build_weights(w_packed, b_packed)
    body = _make_body(l, l_pad, n_groups)

    full = lambda a: pl.BlockSpec(a.shape, lambda i: (0,) * a.ndim)
    out = pl.pallas_call(
        body,
        out_shape=jax.ShapeDtypeStruct((n_steps, rows, l_pad), x_ncl.dtype),
        grid=(n_steps,),
        in_specs=[pl.BlockSpec((1, rows, l_pad), lambda i: (i, 0, 0))]
        + [full(w) for w in ws] + [full(b) for b in bs],
        out_specs=pl.BlockSpec((1, rows, l_pad), lambda i: (i, 0, 0)),
        compiler_params=pltpu.CompilerParams(
            dimension_semantics=("parallel",),
            vmem_limit_bytes=64 * 1024 * 1024,
        ),
    )(xr, *ws, *bs)

    out = out.reshape(n_pad, c, l_pad)
    if n_pad != n or l_pad != l:
        out = out[:n, :, :l]
    return out


# speedup vs baseline: 2.9804x; 1.0790x over previous
import jax
import jax.numpy as jnp
from jax import lax
from jax.experimental import pallas as pl
from jax.experimental.pallas import tpu as pltpu

_LAYER_DIMS = ((3, 16), (16, 32), (32, 16), (16, 3))
_RELUS = (True, True, True, False)
_LANE = 128

_S = 4
_B = 16


def _align(v, m):
    return -(-v // m) * m


def _build_weights(w_packed, b_packed):
    ws = []
    eye = jnp.eye(_S, dtype=jnp.float32)
    n_layers = len(_LAYER_DIMS)
    for li, (cin, cout) in enumerate(_LAYER_DIMS):
        wt = w_packed[3 * li:3 * li + 3, :cout, :cin]
        seg = _align(_S * cin, 16)
        w5 = jnp.einsum('pq,toc->potqc', eye, wt)
        w = w5.reshape(_S * cout, 3, _S * cin)
        w = jnp.pad(w, ((0, 0), (0, 0), (0, seg - _S * cin)))
        w = w.reshape(_S * cout, 3 * seg)
        b = jnp.tile(b_packed[li, :cout, :], (_S, 1))
        w = jnp.concatenate(
            [w, b, jnp.zeros((_S * cout, 15), jnp.float32)], axis=1)
        if li == n_layers - 1:
            slot = 8
            wr = jnp.zeros((slot * _S, w.shape[1]), jnp.float32)
            wr = wr.at[
                (slot * jnp.arange(_S)[:, None]
                 + jnp.arange(cout)[None, :]).reshape(-1)
            ].set(w.reshape(_S * cout, -1))
            w = wr
        else:
            m_pad = _align(_S * cout, 16)
            if m_pad != _S * cout:
                w = jnp.pad(w, ((0, m_pad - _S * cout), (0, 0)))
        ws.append(w.astype(jnp.bfloat16))
    return ws


def _make_body(l_valid, l_pad, n_groups):
    c_in = _LAYER_DIMS[0][0]
    rows_in = _S * c_in
    seg0 = _align(rows_in, 16)
    c_out = _LAYER_DIMS[-1][1]
    bf = jnp.bfloat16

    def body(x_ref, w1, w2, w3, w4, o_ref):
        wrefs = (w1, w2, w3, w4)
        lane = lax.broadcasted_iota(jnp.int32, (16, l_pad), 1)
        sub = lax.broadcasted_iota(jnp.int32, (16, l_pad), 0)
        ones_seg = jnp.where((sub == 0) & (lane < l_valid), 1.0, 0.0).astype(bf)

        def zero_tail(y_rows, relu):
            d = y_rows[:, :l_valid]
            if relu:
                d = jnp.maximum(d, 0.0)
            d = d.astype(bf)
            if l_pad == l_valid:
                return d
            return jnp.concatenate(
                [d, jnp.zeros((d.shape[0], l_pad - l_valid), bf)], axis=1)

        for g in range(n_groups):
            v = x_ref[g * _S:(g + 1) * _S]
            h = v.reshape(rows_in, l_valid)
            h = zero_tail(h, False)
            if seg0 != rows_in:
                h = jnp.concatenate(
                    [h, jnp.zeros((seg0 - rows_in, l_pad), bf)], axis=0)
            for li in range(len(_LAYER_DIMS)):
                prev = pltpu.roll(h, 1, axis=1)
                nxt = pltpu.roll(h, l_pad - 1, axis=1)
                stk = jnp.concatenate([prev, h, nxt, ones_seg], axis=0)
                y = jnp.dot(wrefs[li][...], stk,
                            preferred_element_type=jnp.float32)
                if _RELUS[li]:
                    h = zero_tail(y, True)
                else:
                    for s in range(_S):
                        o_ref[g * _S + s] = y[8 * s:8 * s + c_out, :l_valid]
    return body


def kernel(x_ncl, w_packed, b_packed):
    n, c, l = x_ncl.shape
    l_pad = _align(l + 1, _LANE)
    n_pad = _align(n, _B)

    xp = x_ncl
    if n_pad != n:
        xp = jnp.pad(x_ncl, ((0, n_pad - n), (0, 0), (0, 0)))

    n_steps = n_pad // _B
    n_groups = _B // _S

    ws = _build_weights(w_packed, b_packed)
    body = _make_body(l, l_pad, n_groups)

    full = lambda a: pl.BlockSpec(a.shape, lambda i: (0,) * a.ndim)
    out = pl.pallas_call(
        body,
        out_shape=jax.ShapeDtypeStruct((n_pad, c, l), x_ncl.dtype),
        grid=(n_steps,),
        in_specs=[pl.BlockSpec((_B, c, l), lambda i: (i, 0, 0))]
        + [full(w) for w in ws],
        out_specs=pl.BlockSpec((_B, c, l), lambda i: (i, 0, 0)),
        compiler_params=pltpu.CompilerParams(
            dimension_semantics=("parallel",),
            vmem_limit_bytes=64 * 1024 * 1024,
        ),
    )(xp, *ws)

    if n_pad != n:
        out = out[:n]
    return out


# speedup vs baseline: 3.1148x; 1.0451x over previous
import jax
import jax.numpy as jnp
from jax import lax
from jax.experimental import pallas as pl
from jax.experimental.pallas import tpu as pltpu

_LAYER_DIMS = ((3, 16), (16, 32), (32, 16), (16, 3))
_RELUS = (True, True, True, False)
_LANE = 128

_S = 4
_B = 64


def _align(v, m):
    return -(-v // m) * m


def _build_weights(w_packed, b_packed):
    ws = []
    eye = jnp.eye(_S, dtype=jnp.float32)
    n_layers = len(_LAYER_DIMS)
    for li, (cin, cout) in enumerate(_LAYER_DIMS):
        wt = w_packed[3 * li:3 * li + 3, :cout, :cin]
        seg = _align(_S * cin, 16)
        w5 = jnp.einsum('pq,toc->potqc', eye, wt)
        w = w5.reshape(_S * cout, 3, _S * cin)
        w = jnp.pad(w, ((0, 0), (0, 0), (0, seg - _S * cin)))
        w = w.reshape(_S * cout, 3 * seg)
        b = jnp.tile(b_packed[li, :cout, :], (_S, 1))
        w = jnp.concatenate(
            [w, b, jnp.zeros((_S * cout, 15), jnp.float32)], axis=1)
        if li == n_layers - 1:
            slot = 8
            wr = jnp.zeros((slot * _S, w.shape[1]), jnp.float32)
            wr = wr.at[
                (slot * jnp.arange(_S)[:, None]
                 + jnp.arange(cout)[None, :]).reshape(-1)
            ].set(w.reshape(_S * cout, -1))
            w = wr
        else:
            m_pad = _align(_S * cout, 16)
            if m_pad != _S * cout:
                w = jnp.pad(w, ((0, m_pad - _S * cout), (0, 0)))
        ws.append(w.astype(jnp.bfloat16))
    return ws


def _make_body(l_valid, l_pad, n_groups):
    c_in = _LAYER_DIMS[0][0]
    rows_in = _S * c_in
    seg0 = _align(rows_in, 16)
    c_out = _LAYER_DIMS[-1][1]
    bf = jnp.bfloat16

    def body(x_ref, w1, w2, w3, w4, o_ref):
        wrefs = (w1, w2, w3, w4)
        lane = lax.broadcasted_iota(jnp.int32, (16, l_pad), 1)
        sub = lax.broadcasted_iota(jnp.int32, (16, l_pad), 0)
        ones_seg = jnp.where((sub == 0) & (lane < l_valid), 1.0, 0.0).astype(bf)

        def zero_tail(y_rows, relu):
            d = y_rows[:, :l_valid]
            if relu:
                d = jnp.maximum(d, 0.0)
            d = d.astype(bf)
            if l_pad == l_valid:
                return d
            return jnp.concatenate(
                [d, jnp.zeros((d.shape[0], l_pad - l_valid), bf)], axis=1)

        for g in range(n_groups):
            v = x_ref[g * _S:(g + 1) * _S]
            h = v.reshape(rows_in, l_valid)
            h = zero_tail(h, False)
            if seg0 != rows_in:
                h = jnp.concatenate(
                    [h, jnp.zeros((seg0 - rows_in, l_pad), bf)], axis=0)
            for li in range(len(_LAYER_DIMS)):
                prev = pltpu.roll(h, 1, axis=1)
                nxt = pltpu.roll(h, l_pad - 1, axis=1)
                stk = jnp.concatenate([prev, h, nxt, ones_seg], axis=0)
                y = jnp.dot(wrefs[li][...], stk,
                            preferred_element_type=jnp.float32)
                if _RELUS[li]:
                    h = zero_tail(y, True)
                else:
                    for s in range(_S):
                        o_ref[g * _S + s] = y[8 * s:8 * s + c_out, :l_valid]
    return body


def kernel(x_ncl, w_packed, b_packed):
    n, c, l = x_ncl.shape
    l_pad = _align(l + 1, _LANE)
    n_pad = _align(n, _B)

    xp = x_ncl
    if n_pad != n:
        xp = jnp.pad(x_ncl, ((0, n_pad - n), (0, 0), (0, 0)))

    n_steps = n_pad // _B
    n_groups = _B // _S

    ws = _build_weights(w_packed, b_packed)
    body = _make_body(l, l_pad, n_groups)

    full = lambda a: pl.BlockSpec(a.shape, lambda i: (0,) * a.ndim)
    out = pl.pallas_call(
        body,
        out_shape=jax.ShapeDtypeStruct((n_pad, c, l), x_ncl.dtype),
        grid=(n_steps,),
        in_specs=[pl.BlockSpec((_B, c, l), lambda i: (i, 0, 0))]
        + [full(w) for w in ws],
        out_specs=pl.BlockSpec((_B, c, l), lambda i: (i, 0, 0)),
        compiler_params=pltpu.CompilerParams(
            dimension_semantics=("parallel",),
            vmem_limit_bytes=64 * 1024 * 1024,
        ),
    )(xp, *ws)

    if n_pad != n:
        out = out[:n]
    return out


# speedup vs baseline: 3.9081x; 1.2547x over previous
import jax
import jax.numpy as jnp
from jax import lax
from jax.experimental import pallas as pl
from jax.experimental.pallas import tpu as pltpu

_LAYER_DIMS = ((3, 16), (16, 32), (32, 16), (16, 3))
_RELUS = (True, True, True, False)
_LANE = 128
_F = 4
_B = 64


def _align(v, m):
    return -(-v // m) * m


def _build_weights(w_packed, b_packed):
    wms, wes = [], []
    for li, (cin, cout) in enumerate(_LAYER_DIMS):
        wt = w_packed[3 * li:3 * li + 3, :cout, :cin]
        last = li == len(_LAYER_DIMS) - 1
        om = 3 if last else cout
        m = _align(_F * om, 16)
        kh = _align(_F * cin, 16)
        seg = kh if li == 0 else cin
        wm = jnp.zeros((m, kh), jnp.float32)
        we = jnp.zeros((m, 2 * seg + 16), jnp.float32)
        for r in range(_F):
            ro = r * om
            for t in range(3):
                rp = r + t - 1
                if 0 <= rp < _F:
                    wm = wm.at[ro:ro + cout,
                               rp * cin:rp * cin + cin].set(wt[t])
            if r == 0:
                off = (_F - 1) * cin if li == 0 else 0
                we = we.at[ro:ro + cout, off:off + cin].set(wt[0])
            if r == _F - 1:
                off = seg
                we = we.at[ro:ro + cout, off:off + cin].set(wt[2])
        bcol = jnp.tile(b_packed[li, :cout, 0], (_F,))
        rows = (om * jnp.arange(_F)[:, None]
                + jnp.arange(cout)[None, :]).reshape(-1)
        we = we.at[rows, 2 * seg].set(bcol)
        wms.append(wm.astype(jnp.bfloat16))
        wes.append(we.astype(jnp.bfloat16))
    return wms, wes


def _make_body(q_lanes, n_block):
    cin1 = _LAYER_DIMS[0][0]
    rows_in = _F * cin1
    bf = jnp.bfloat16
    width = n_block * q_lanes

    def body(x_ref, wm1, wm2, wm3, wm4, we1, we2, we3, we4, o_ref):
        wms = (wm1, wm2, wm3, wm4)
        wes = (we1, we2, we3, we4)
        sub16 = lax.broadcasted_iota(jnp.int32, (16, width), 0)
        lane16 = lax.broadcasted_iota(jnp.int32, (16, width), 1)
        ones_seg = jnp.where(
            (sub16 == 0) & (lane16 >= 0), 1.0, 0.0).astype(bf)

        def qpos(shape):
            lane = lax.broadcasted_iota(jnp.int32, shape, 1)
            return lax.bitwise_and(lane, q_lanes - 1)

        def shift_r(a):
            zc = jnp.zeros((a.shape[0], 1), bf)
            s = jnp.concatenate([zc, a[:, :-1]], axis=1)
            return jnp.where(qpos(a.shape) == 0, jnp.bfloat16(0), s)

        def shift_l(a):
            zc = jnp.zeros((a.shape[0], 1), bf)
            s = jnp.concatenate([a[:, 1:], zc], axis=1)
            return jnp.where(qpos(a.shape) == q_lanes - 1, jnp.bfloat16(0), s)

        h = x_ref[0].astype(bf)
        h = jnp.concatenate(
            [h, jnp.zeros((16 - rows_in, width), bf)], axis=0)
        for li in range(len(_LAYER_DIMS)):
            cin = _LAYER_DIMS[li][0]
            if li == 0:
                ep, en = shift_r(h), shift_l(h)
            else:
                ep = shift_r(h[(_F - 1) * cin:_F * cin])
                en = shift_l(h[:cin])
            edges = jnp.concatenate([ep, en, ones_seg], axis=0)
            y = (jnp.dot(wms[li][...], h, preferred_element_type=jnp.float32)
                 + jnp.dot(wes[li][...], edges,
                           preferred_element_type=jnp.float32))
            if _RELUS[li]:
                h = jnp.maximum(y, 0.0).astype(bf)
            else:
                o_ref[0] = y[:_F * 3]
    return body


def kernel(x_ncl, w_packed, b_packed):
    n, c, l = x_ncl.shape
    l_pad = _align(l, _F * _LANE)
    n_pad = _align(n, _B)

    xp = x_ncl
    if n_pad != n or l_pad != l:
        xp = jnp.pad(x_ncl, ((0, n_pad - n), (0, 0), (0, l_pad - l)))

    q = l_pad // _F
    n_steps = n_pad // _B
    width = _B * q
    xt = (xp.reshape(n_steps, _B, c, q, _F)
          .transpose(0, 4, 2, 1, 3)
          .reshape(n_steps, _F * c, width))

    wms, wes = _build_weights(w_packed, b_packed)
    body = _make_body(q, _B)

    full = lambda a: pl.BlockSpec(a.shape, lambda i: (0,) * a.ndim)
    out = pl.pallas_call(
        body,
        out_shape=jax.ShapeDtypeStruct((n_steps, _F * c, width), x_ncl.dtype),
        grid=(n_steps,),
        in_specs=[pl.BlockSpec((1, _F * c, width), lambda i: (i, 0, 0))]
        + [full(w) for w in wms] + [full(w) for w in wes],
        out_specs=pl.BlockSpec((1, _F * c, width), lambda i: (i, 0, 0)),
        compiler_params=pltpu.CompilerParams(
            dimension_semantics=("parallel",),
            vmem_limit_bytes=100 * 1024 * 1024,
        ),
    )(xt, *wms, *wes)

    out = (out.reshape(n_pad // _B, _F, c, _B, q)
           .transpose(0, 3, 2, 4, 1)
           .reshape(n_pad, c, l_pad))
    if n_pad != n or l_pad != l:
        out = out[:n, :, :l]
    return out


# speedup vs baseline: 4.4861x; 1.1479x over previous
import jax
import jax.numpy as jnp
from jax import lax
from jax.experimental import pallas as pl
from jax.experimental.pallas import tpu as pltpu

_LAYER_DIMS = ((3, 16), (16, 32), (32, 16), (16, 3))
_RELUS = (True, True, True, False)
_LANE = 128
_F = 4
_B = 128


def _align(v, m):
    return -(-v // m) * m


def _build_weights(w_packed, b_packed):
    wms, wes = [], []
    for li, (cin, cout) in enumerate(_LAYER_DIMS):
        wt = w_packed[3 * li:3 * li + 3, :cout, :cin]
        last = li == len(_LAYER_DIMS) - 1
        om = 3 if last else cout
        m = _align(_F * om, 16)
        kh = _align(_F * cin, 16)
        seg = kh if li == 0 else cin
        wm = jnp.zeros((m, kh), jnp.float32)
        we = jnp.zeros((m, 2 * seg + 16), jnp.float32)
        for r in range(_F):
            ro = r * om
            for t in range(3):
                rp = r + t - 1
                if 0 <= rp < _F:
                    wm = wm.at[ro:ro + cout,
                               rp * cin:rp * cin + cin].set(wt[t])
            if r == 0:
                off = (_F - 1) * cin if li == 0 else 0
                we = we.at[ro:ro + cout, off:off + cin].set(wt[0])
            if r == _F - 1:
                off = seg
                we = we.at[ro:ro + cout, off:off + cin].set(wt[2])
        bcol = jnp.tile(b_packed[li, :cout, 0], (_F,))
        rows = (om * jnp.arange(_F)[:, None]
                + jnp.arange(cout)[None, :]).reshape(-1)
        we = we.at[rows, 2 * seg].set(bcol)
        wms.append(wm.astype(jnp.bfloat16))
        wes.append(we.astype(jnp.bfloat16))
    return wms, wes


def _make_body(width):
    cin1 = _LAYER_DIMS[0][0]
    rows_in = _F * cin1
    bf = jnp.bfloat16

    def shift_r(a):
        zc = jnp.zeros((a.shape[0], _B), bf)
        return jnp.concatenate([zc, a[:, :-_B]], axis=1)

    def shift_l(a):
        zc = jnp.zeros((a.shape[0], _B), bf)
        return jnp.concatenate([a[:, _B:], zc], axis=1)

    def body(x_ref, wm1, wm2, wm3, wm4, we1, we2, we3, we4, o_ref):
        wms = (wm1, wm2, wm3, wm4)
        wes = (we1, we2, we3, we4)
        sub16 = lax.broadcasted_iota(jnp.int32, (16, width), 0)
        lane16 = lax.broadcasted_iota(jnp.int32, (16, width), 1)
        ones_seg = jnp.where(
            (sub16 == 0) & (lane16 >= 0), 1.0, 0.0).astype(bf)

        h = x_ref[0]
        h = jnp.concatenate(
            [h, jnp.zeros((16 - rows_in, width), bf)], axis=0)
        for li in range(len(_LAYER_DIMS)):
            cin = _LAYER_DIMS[li][0]
            if li == 0:
                ep, en = shift_r(h), shift_l(h)
            else:
                ep = shift_r(h[(_F - 1) * cin:_F * cin])
                en = shift_l(h[:cin])
            edges = jnp.concatenate([ep, en, ones_seg], axis=0)
            y = (jnp.dot(wms[li][...], h, preferred_element_type=jnp.float32)
                 + jnp.dot(wes[li][...], edges,
                           preferred_element_type=jnp.float32))
            if _RELUS[li]:
                h = jnp.maximum(y, 0.0).astype(bf)
            else:
                o_ref[0] = y[:_F * 3].astype(bf)
    return body


def kernel(x_ncl, w_packed, b_packed):
    n, c, l = x_ncl.shape
    l_pad = _align(l, _F)
    n_pad = _align(n, _B)

    xp = x_ncl
    if n_pad != n or l_pad != l:
        xp = jnp.pad(x_ncl, ((0, n_pad - n), (0, 0), (0, l_pad - l)))

    q = l_pad // _F
    n_steps = n_pad // _B
    width = q * _B
    xt = (xp.astype(jnp.bfloat16)
          .reshape(n_steps, _B, c, q, _F)
          .transpose(0, 4, 2, 3, 1)
          .reshape(n_steps, _F * c, width))

    wms, wes = _build_weights(w_packed, b_packed)
    body = _make_body(width)

    full = lambda a: pl.BlockSpec(a.shape, lambda i: (0,) * a.ndim)
    out = pl.pallas_call(
        body,
        out_shape=jax.ShapeDtypeStruct((n_steps, _F * c, width), jnp.bfloat16),
        grid=(n_steps,),
        in_specs=[pl.BlockSpec((1, _F * c, width), lambda i: (i, 0, 0))]
        + [full(w) for w in wms] + [full(w) for w in wes],
        out_specs=pl.BlockSpec((1, _F * c, width), lambda i: (i, 0, 0)),
        compiler_params=pltpu.CompilerParams(
            dimension_semantics=("parallel",),
            vmem_limit_bytes=100 * 1024 * 1024,
        ),
    )(xt, *wms, *wes)

    out = (out.reshape(n_steps, _F, c, q, _B)
           .transpose(0, 4, 2, 3, 1)
           .reshape(n_pad, c, l_pad)
           .astype(x_ncl.dtype))
    if n_pad != n or l_pad != l:
        out = out[:n, :, :l]
    return out
